# baseline (speedup 1.0000x reference)
import functools

import jax
import jax.numpy as jnp
from jax import lax
from jax.experimental import pallas as pl
from jax.experimental.pallas import tpu as pltpu
from jax.experimental.pallas import tpu_sc as plsc

_B, _T, _F = 64, 2048, 512
_HF = 256
_ROWS = _B * _T
_NC, _NS = 2, 16
_NW = _NC * _NS
_RPW = _ROWS // _NW
_CHUNK = 32
_NCHUNK = _RPW // _CHUNK

_KS2 = 0x1BD11BDA
_ROT = ((13, 15, 26, 6), (17, 29, 16, 24))


def _u32(v):
    return jnp.uint32(v)


def _rotl(x, r):
    return (x << _u32(r)) | (x >> _u32(32 - r))


def _threefry_bits(lo):
    x0 = jnp.zeros((16,), jnp.uint32)
    x1 = lo
    for i in range(5):
        for r in _ROT[i % 2]:
            x0 = x0 + x1
            x1 = _rotl(x1, r)
            x1 = x1 ^ x0
        ks_a = (0, 0, _KS2)[(i + 1) % 3]
        ks_b = (0, 0, _KS2)[(i + 2) % 3]
        if ks_a:
            x0 = x0 + _u32(ks_a)
        x1 = x1 + _u32(ks_b + i + 1)
    return x0 ^ x1


def _sort_units_multi(rows):
    def vs(a, desc):
        sk, _ = plsc.sort_key_val(a, a, descending=desc)
        return sk

    for i in range(16):
        for un in rows:
            un[i] = vs(un[i], (i & 1) == 1)
    for ku in (2, 4, 8, 16):
        su = ku // 2
        while su >= 1:
            for un in rows:
                for i in range(16):
                    p = i ^ su
                    if p > i:
                        mn = jnp.minimum(un[i], un[p])
                        mx = jnp.maximum(un[i], un[p])
                        if (i & ku) == 0:
                            un[i], un[p] = mn, mx
                        else:
                            un[i], un[p] = mx, mn
            su //= 2
        for i in range(16):
            for un in rows:
                un[i] = vs(un[i], (i & ku) != 0)
    return rows


_mesh = plsc.VectorSubcoreMesh(
    core_axis_name="c", subcore_axis_name="s",
    num_cores=_NC, num_subcores=_NS,
)


@functools.partial(
    pl.kernel,
    out_type=jax.ShapeDtypeStruct((_ROWS, _F), jnp.float32),
    mesh=_mesh,
    scratch_types=[
        pltpu.VMEM((_CHUNK, _F), jnp.float32),
        pltpu.VMEM((_CHUNK, _F), jnp.float32),
    ],
    compiler_params=pltpu.CompilerParams(needs_layout_passes=False),
)
def _sc_permute(x_hbm, out_hbm, xbuf, obuf):
    wid = lax.axis_index("s") * _NC + lax.axis_index("c")
    lane = lax.iota(jnp.int32, 16)

    def chunk_body(c, carry):
        row0 = wid * _RPW + c * _CHUNK
        pltpu.sync_copy(x_hbm.at[pl.ds(row0, _CHUNK)], xbuf)

        def pair_body(rr, rcarry):
            rws = [rr * 2, rr * 2 + 1]
            rows = []
            for r in rws:
                fbase = (row0 + r).astype(jnp.uint32) * _u32(_HF)
                un = []
                for u in range(16):
                    jvec = (u * 16 + lane).astype(jnp.uint32)
                    bits = _threefry_bits(fbase + jvec)
                    un.append(((bits >> _u32(1)) & _u32(0xFFFFFF00)) | jvec)
                rows.append(un)

            rows = _sort_units_multi(rows)

            for r, un in zip(rws, rows):
                rvec = jnp.zeros((16,), jnp.int32) + r
                for i in range(16):
                    obuf[r, pl.ds(i * 16, 16)] = xbuf[r, pl.ds(i * 16, 16)]
                for i in range(16):
                    idx = (un[i] & _u32(0xFF)).astype(jnp.int32) + _HF
                    vals = plsc.load_gather(xbuf, [rvec, idx])
                    obuf[r, pl.ds(_HF + i * 16, 16)] = vals
            return rcarry

        lax.fori_loop(0, _CHUNK // 2, pair_body, 0)
        pltpu.sync_copy(obuf, out_hbm.at[pl.ds(row0, _CHUNK)])
        return carry

    lax.fori_loop(0, _NCHUNK, chunk_body, 0)


def kernel(x):
    B, T, F = x.shape
    out = _sc_permute(x.reshape(B * T, F))
    return out.reshape(B, T, F)


# speedup vs baseline: 3.3011x; 3.3011x over previous
import functools

import jax
import jax.numpy as jnp
from jax import lax
from jax.experimental import pallas as pl
from jax.experimental.pallas import tpu as pltpu
from jax.experimental.pallas import tpu_sc as plsc

_B, _T, _F = 64, 2048, 512
_HF = 256
_ROWS = _B * _T
_NC, _NS = 2, 16
_NW = _NC * _NS
_RPW = _ROWS // _NW
_CHUNK = 32
_NCHUNK = _RPW // _CHUNK
_RB = 512

_KS2 = 0x1BD11BDA
_ROT = ((13, 15, 26, 6), (17, 29, 16, 24))


def _u32(v):
    return jnp.uint32(v)


def _rotl(x, r):
    return (x << _u32(r)) | (x >> _u32(32 - r))


def _threefry_bits(lo):
    x0 = jnp.zeros_like(lo)
    x1 = lo
    for i in range(5):
        for r in _ROT[i % 2]:
            x0 = x0 + x1
            x1 = _rotl(x1, r)
            x1 = x1 ^ x0
        ks_a = (0, 0, _KS2)[(i + 1) % 3]
        ks_b = (0, 0, _KS2)[(i + 2) % 3]
        if ks_a:
            x0 = x0 + _u32(ks_a)
        x1 = x1 + _u32(ks_b + i + 1)
    return x0 ^ x1


def _tc_keys_body(k_ref):
    b = pl.program_id(0)
    rows = lax.broadcasted_iota(jnp.uint32, (_RB, _HF), 0)
    cols = lax.broadcasted_iota(jnp.uint32, (_RB, _HF), 1)
    f = ((b * _RB).astype(jnp.uint32) + rows) * _u32(_HF) + cols
    bits = _threefry_bits(f)
    k_ref[...] = ((bits >> _u32(1)) & _u32(0xFFFFFF00)) | cols


def _tc_keys():
    return pl.pallas_call(
        _tc_keys_body,
        out_shape=jax.ShapeDtypeStruct((_ROWS, _HF), jnp.uint32),
        grid=(_ROWS // _RB,),
        out_specs=pl.BlockSpec((_RB, _HF), lambda b: (b, 0)),
    )()


def _sort_units_multi(rows):
    def vs(a, desc):
        sk, _ = plsc.sort_key_val(a, a, descending=desc)
        return sk

    for i in range(16):
        for un in rows:
            un[i] = vs(un[i], (i & 1) == 1)
    for ku in (2, 4, 8, 16):
        su = ku // 2
        while su >= 1:
            for un in rows:
                for i in range(16):
                    p = i ^ su
                    if p > i:
                        mn = jnp.minimum(un[i], un[p])
                        mx = jnp.maximum(un[i], un[p])
                        if (i & ku) == 0:
                            un[i], un[p] = mn, mx
                        else:
                            un[i], un[p] = mx, mn
            su //= 2
        for i in range(16):
            for un in rows:
                un[i] = vs(un[i], (i & ku) != 0)
    return rows


_mesh = plsc.VectorSubcoreMesh(
    core_axis_name="c", subcore_axis_name="s",
    num_cores=_NC, num_subcores=_NS,
)


@functools.partial(
    pl.kernel,
    out_type=jax.ShapeDtypeStruct((_ROWS, _F), jnp.float32),
    mesh=_mesh,
    scratch_types=[
        pltpu.VMEM((_CHUNK, _F), jnp.float32),
        pltpu.VMEM((_CHUNK, _F), jnp.float32),
        pltpu.VMEM((_CHUNK, _HF), jnp.uint32),
    ],
    compiler_params=pltpu.CompilerParams(needs_layout_passes=False),
)
def _sc_permute(x_hbm, keys_hbm, out_hbm, xbuf, obuf, kbuf):
    wid = lax.axis_index("s") * _NC + lax.axis_index("c")

    def chunk_body(c, carry):
        row0 = wid * _RPW + c * _CHUNK
        pltpu.sync_copy(x_hbm.at[pl.ds(row0, _CHUNK)], xbuf)
        pltpu.sync_copy(keys_hbm.at[pl.ds(row0, _CHUNK)], kbuf)

        def pair_body(rr, rcarry):
            rws = [rr * 2, rr * 2 + 1]
            rows = [[kbuf[r, pl.ds(i * 16, 16)] for i in range(16)]
                    for r in rws]
            rows = _sort_units_multi(rows)
            for r, un in zip(rws, rows):
                rvec = jnp.zeros((16,), jnp.int32) + r
                for i in range(16):
                    obuf[r, pl.ds(i * 16, 16)] = xbuf[r, pl.ds(i * 16, 16)]
                for i in range(16):
                    idx = (un[i] & _u32(0xFF)).astype(jnp.int32) + _HF
                    vals = plsc.load_gather(xbuf, [rvec, idx])
                    obuf[r, pl.ds(_HF + i * 16, 16)] = vals
            return rcarry

        lax.fori_loop(0, _CHUNK // 2, pair_body, 0)
        pltpu.sync_copy(obuf, out_hbm.at[pl.ds(row0, _CHUNK)])
        return carry

    lax.fori_loop(0, _NCHUNK, chunk_body, 0)


def kernel(x):
    B, T, F = x.shape
    keys = _tc_keys()
    out = _sc_permute(x.reshape(B * T, F), keys)
    return out.reshape(B, T, F)


# speedup vs baseline: 4.3232x; 1.3096x over previous
import functools

import jax
import jax.numpy as jnp
from jax import lax
from jax.experimental import pallas as pl
from jax.experimental.pallas import tpu as pltpu
from jax.experimental.pallas import tpu_sc as plsc

_B, _T, _F = 64, 2048, 512
_HF = 256
_ROWS = _B * _T
_NC, _NS = 2, 16
_NW = _NC * _NS
_RPW = _ROWS // _NW
_CHUNK = 32
_NCHUNK = _RPW // _CHUNK
_RB = 512

_KS2 = 0x1BD11BDA
_ROT = ((13, 15, 26, 6), (17, 29, 16, 24))


def _u32(v):
    return jnp.uint32(v)


def _rotl(x, r):
    return (x << _u32(r)) | (x >> _u32(32 - r))


def _threefry_bits(lo):
    x0 = jnp.zeros_like(lo)
    x1 = lo
    for i in range(5):
        for r in _ROT[i % 2]:
            x0 = x0 + x1
            x1 = _rotl(x1, r)
            x1 = x1 ^ x0
        ks_a = (0, 0, _KS2)[(i + 1) % 3]
        ks_b = (0, 0, _KS2)[(i + 2) % 3]
        if ks_a:
            x0 = x0 + _u32(ks_a)
        x1 = x1 + _u32(ks_b + i + 1)
    return x0 ^ x1


def _tc_keys_body(k_ref):
    b = pl.program_id(0)
    rows = lax.broadcasted_iota(jnp.uint32, (_RB, _HF), 0)
    cols = lax.broadcasted_iota(jnp.uint32, (_RB, _HF), 1)
    f = ((b * _RB).astype(jnp.uint32) + rows) * _u32(_HF) + cols
    bits = _threefry_bits(f)
    k_ref[...] = ((bits >> _u32(1)) & _u32(0xFFFFFF00)) | cols


def _tc_keys():
    return pl.pallas_call(
        _tc_keys_body,
        out_shape=jax.ShapeDtypeStruct((_ROWS, _HF), jnp.uint32),
        grid=(_ROWS // _RB,),
        out_specs=pl.BlockSpec((_RB, _HF), lambda b: (b, 0)),
    )()


def _sort_units_multi(rows):
    def vs(a, desc):
        sk, _ = plsc.sort_key_val(a, a, descending=desc)
        return sk

    for i in range(16):
        for un in rows:
            un[i] = vs(un[i], (i & 1) == 1)
    for ku in (2, 4, 8, 16):
        su = ku // 2
        while su >= 1:
            for un in rows:
                for i in range(16):
                    p = i ^ su
                    if p > i:
                        mn = jnp.minimum(un[i], un[p])
                        mx = jnp.maximum(un[i], un[p])
                        if (i & ku) == 0:
                            un[i], un[p] = mn, mx
                        else:
                            un[i], un[p] = mx, mn
            su //= 2
        for i in range(16):
            for un in rows:
                un[i] = vs(un[i], (i & ku) != 0)
    return rows


_mesh = plsc.VectorSubcoreMesh(
    core_axis_name="c", subcore_axis_name="s",
    num_cores=_NC, num_subcores=_NS,
)


@functools.partial(
    pl.kernel,
    out_type=jax.ShapeDtypeStruct((_ROWS, _F), jnp.float32),
    mesh=_mesh,
    scratch_types=[
        pltpu.VMEM((2, _CHUNK, _F), jnp.float32),
        pltpu.VMEM((2, _CHUNK, _F), jnp.float32),
        pltpu.VMEM((2, _CHUNK, _HF), jnp.uint32),
        pltpu.SemaphoreType.DMA,
        pltpu.SemaphoreType.DMA,
        pltpu.SemaphoreType.DMA,
        pltpu.SemaphoreType.DMA,
        pltpu.SemaphoreType.DMA,
        pltpu.SemaphoreType.DMA,
    ],
    compiler_params=pltpu.CompilerParams(needs_layout_passes=False),
)
def _sc_permute(x_hbm, keys_hbm, out_hbm, xbuf, obuf, kbuf,
                sx0, sx1, sk0, sk1, ss0, ss1):
    wid = lax.axis_index("s") * _NC + lax.axis_index("c")
    sx = [sx0, sx1]
    sk = [sk0, sk1]
    ss = [ss0, ss1]

    def row0_of(ch):
        return wid * _RPW + ch * _CHUNK

    def ld(ch, b):
        pltpu.make_async_copy(
            x_hbm.at[pl.ds(row0_of(ch), _CHUNK)], xbuf.at[b], sx[b]).start()
        pltpu.make_async_copy(
            keys_hbm.at[pl.ds(row0_of(ch), _CHUNK)], kbuf.at[b], sk[b]).start()

    def ld_wait(ch, b):
        pltpu.make_async_copy(
            x_hbm.at[pl.ds(row0_of(ch), _CHUNK)], xbuf.at[b], sx[b]).wait()
        pltpu.make_async_copy(
            keys_hbm.at[pl.ds(row0_of(ch), _CHUNK)], kbuf.at[b], sk[b]).wait()

    def st(ch, b):
        pltpu.make_async_copy(
            obuf.at[b], out_hbm.at[pl.ds(row0_of(ch), _CHUNK)], ss[b]).start()

    def st_wait(ch, b):
        pltpu.make_async_copy(
            obuf.at[b], out_hbm.at[pl.ds(row0_of(ch), _CHUNK)], ss[b]).wait()

    ld(0, 0)

    def chunk2_body(cc, carry):
        for b in range(2):
            ch = cc * 2 + b
            ld_wait(ch, b)

            @pl.when(ch + 1 < _NCHUNK)
            def _():
                ld(ch + 1, 1 - b)

            @pl.when(ch >= 2)
            def _():
                st_wait(ch - 2, b)

            def pair_body(rr, rcarry):
                rws = [rr * 2, rr * 2 + 1]
                rows = [[kbuf[b, r, pl.ds(i * 16, 16)] for i in range(16)]
                        for r in rws]
                rows = _sort_units_multi(rows)
                for r, un in zip(rws, rows):
                    rvec = jnp.zeros((16,), jnp.int32) + r
                    for i in range(16):
                        obuf[b, r, pl.ds(i * 16, 16)] = \
                            xbuf[b, r, pl.ds(i * 16, 16)]
                    for i in range(16):
                        idx = (un[i] & _u32(0xFF)).astype(jnp.int32) + _HF
                        vals = plsc.load_gather(
                            xbuf.at[b], [rvec, idx])
                        obuf[b, r, pl.ds(_HF + i * 16, 16)] = vals
                return rcarry

            lax.fori_loop(0, _CHUNK // 2, pair_body, 0)
            st(ch, b)
        return carry

    lax.fori_loop(0, _NCHUNK // 2, chunk2_body, 0)
    st_wait(_NCHUNK - 2, 0)
    st_wait(_NCHUNK - 1, 1)


def kernel(x):
    B, T, F = x.shape
    keys = _tc_keys()
    out = _sc_permute(x.reshape(B * T, F), keys)
    return out.reshape(B, T, F)


# speedup vs baseline: 6.1607x; 1.4250x over previous
import functools

import jax
import jax.numpy as jnp
from jax import lax
from jax.experimental import pallas as pl
from jax.experimental.pallas import tpu as pltpu
from jax.experimental.pallas import tpu_sc as plsc

_B, _T, _F = 64, 2048, 512
_HF = 256
_ROWS = _B * _T
_NC, _NS = 2, 16
_NW = _NC * _NS
_NG = 8
_GROWS = _ROWS // _NG
_RPW = _GROWS // _NW
_CHUNK = 32
_NCHUNK = _RPW // _CHUNK
_RB = 512

_KS2 = 0x1BD11BDA
_ROT = ((13, 15, 26, 6), (17, 29, 16, 24))


def _u32(v):
    return jnp.uint32(v)


def _rotl(x, r):
    return (x << _u32(r)) | (x >> _u32(32 - r))


def _threefry_bits(lo):
    x0 = jnp.zeros_like(lo)
    x1 = lo
    for i in range(5):
        for r in _ROT[i % 2]:
            x0 = x0 + x1
            x1 = _rotl(x1, r)
            x1 = x1 ^ x0
        ks_a = (0, 0, _KS2)[(i + 1) % 3]
        ks_b = (0, 0, _KS2)[(i + 2) % 3]
        if ks_a:
            x0 = x0 + _u32(ks_a)
        x1 = x1 + _u32(ks_b + i + 1)
    return x0 ^ x1


def _make_tc_keys(group):
    def body(k_ref):
        b = pl.program_id(0)
        rows = lax.broadcasted_iota(jnp.uint32, (_RB, _HF), 0)
        cols = lax.broadcasted_iota(jnp.uint32, (_RB, _HF), 1)
        f = (_u32(group * _GROWS) + (b * _RB).astype(jnp.uint32) + rows) \
            * _u32(_HF) + cols
        bits = _threefry_bits(f)
        k_ref[...] = ((bits >> _u32(1)) & _u32(0xFFFFFF00)) | cols

    return pl.pallas_call(
        body,
        out_shape=jax.ShapeDtypeStruct((_GROWS, _HF), jnp.uint32),
        grid=(_GROWS // _RB,),
        out_specs=pl.BlockSpec((_RB, _HF), lambda b: (b, 0)),
    )


def _sort_units_multi(rows):
    def vs(a, desc):
        sk, _ = plsc.sort_key_val(a, a, descending=desc)
        return sk

    for i in range(16):
        for un in rows:
            un[i] = vs(un[i], (i & 1) == 1)
    for ku in (2, 4, 8, 16):
        su = ku // 2
        while su >= 1:
            for un in rows:
                for i in range(16):
                    p = i ^ su
                    if p > i:
                        mn = jnp.minimum(un[i], un[p])
                        mx = jnp.maximum(un[i], un[p])
                        if (i & ku) == 0:
                            un[i], un[p] = mn, mx
                        else:
                            un[i], un[p] = mx, mn
            su //= 2
        for i in range(16):
            for un in rows:
                un[i] = vs(un[i], (i & ku) != 0)
    return rows


_mesh = plsc.VectorSubcoreMesh(
    core_axis_name="c", subcore_axis_name="s",
    num_cores=_NC, num_subcores=_NS,
)


def _make_sc_permute(group):
    @functools.partial(
        pl.kernel,
        out_type=jax.ShapeDtypeStruct((_GROWS, _F), jnp.float32),
        mesh=_mesh,
        scratch_types=[
            pltpu.VMEM((2, _CHUNK, _F), jnp.float32),
            pltpu.VMEM((2, _CHUNK, _F), jnp.float32),
            pltpu.VMEM((2, _CHUNK, _HF), jnp.uint32),
            pltpu.SemaphoreType.DMA,
            pltpu.SemaphoreType.DMA,
            pltpu.SemaphoreType.DMA,
            pltpu.SemaphoreType.DMA,
            pltpu.SemaphoreType.DMA,
            pltpu.SemaphoreType.DMA,
        ],
        compiler_params=pltpu.CompilerParams(needs_layout_passes=False),
    )
    def sc_permute(x_hbm, keys_hbm, out_hbm, xbuf, obuf, kbuf,
                   sx0, sx1, sk0, sk1, ss0, ss1):
        wid = lax.axis_index("s") * _NC + lax.axis_index("c")
        sx = [sx0, sx1]
        sk = [sk0, sk1]
        ss = [ss0, ss1]

        def row0_of(ch):
            return wid * _RPW + ch * _CHUNK

        def ld(ch, b):
            pltpu.make_async_copy(
                x_hbm.at[pl.ds(group * _GROWS + row0_of(ch), _CHUNK)],
                xbuf.at[b], sx[b]).start()
            pltpu.make_async_copy(
                keys_hbm.at[pl.ds(row0_of(ch), _CHUNK)],
                kbuf.at[b], sk[b]).start()

        def ld_wait(ch, b):
            pltpu.make_async_copy(
                x_hbm.at[pl.ds(group * _GROWS + row0_of(ch), _CHUNK)],
                xbuf.at[b], sx[b]).wait()
            pltpu.make_async_copy(
                keys_hbm.at[pl.ds(row0_of(ch), _CHUNK)],
                kbuf.at[b], sk[b]).wait()

        def st(ch, b):
            pltpu.make_async_copy(
                obuf.at[b], out_hbm.at[pl.ds(row0_of(ch), _CHUNK)],
                ss[b]).start()

        def st_wait(ch, b):
            pltpu.make_async_copy(
                obuf.at[b], out_hbm.at[pl.ds(row0_of(ch), _CHUNK)],
                ss[b]).wait()

        ld(0, 0)

        def chunk2_body(cc, carry):
            for b in range(2):
                ch = cc * 2 + b
                ld_wait(ch, b)

                @pl.when(ch + 1 < _NCHUNK)
                def _():
                    ld(ch + 1, 1 - b)

                @pl.when(ch >= 2)
                def _():
                    st_wait(ch - 2, b)

                def pair_body(rr, rcarry):
                    rws = [rr * 2, rr * 2 + 1]
                    rows = [[kbuf[b, r, pl.ds(i * 16, 16)] for i in range(16)]
                            for r in rws]
                    rows = _sort_units_multi(rows)
                    for r, un in zip(rws, rows):
                        rvec = jnp.zeros((16,), jnp.int32) + r
                        for i in range(16):
                            obuf[b, r, pl.ds(i * 16, 16)] = \
                                xbuf[b, r, pl.ds(i * 16, 16)]
                        for i in range(16):
                            idx = (un[i] & _u32(0xFF)).astype(jnp.int32) + _HF
                            vals = plsc.load_gather(
                                xbuf.at[b], [rvec, idx])
                            obuf[b, r, pl.ds(_HF + i * 16, 16)] = vals
                    return rcarry

                lax.fori_loop(0, _CHUNK // 2, pair_body, 0)
                st(ch, b)
            return carry

        lax.fori_loop(0, _NCHUNK // 2, chunk2_body, 0)
        st_wait(_NCHUNK - 2, 0)
        st_wait(_NCHUNK - 1, 1)

    return sc_permute


_TC_KEYS = [_make_tc_keys(g) for g in range(_NG)]
_SC_PERMUTE = [_make_sc_permute(g) for g in range(_NG)]


def kernel(x):
    B, T, F = x.shape
    x2 = x.reshape(B * T, F)
    outs = []
    for g in range(_NG):
        keys_g = _TC_KEYS[g]()
        outs.append(_SC_PERMUTE[g](x2, keys_g))
    return jnp.concatenate(outs, axis=0).reshape(B, T, F)
